# baseline (speedup 1.0000x reference)
import functools

import jax
import jax.numpy as jnp
from jax import lax
from jax.experimental import pallas as pl
from jax.experimental.pallas import tpu as pltpu
from jax.experimental.pallas import tpu_sc as plsc

N = 10000
E = 160000
D_IN = 128
D_EDGE = 16
H = 32
EH = 128
STEPS = 3

NC = 2
NS = 16
NW = NC * NS
IB = 125
NSTR = 8
SUP = IB * NSTR
NSUP = E // SUP
SUPW = NSUP // NW
ROWS_PER_SUB = N // NS


def _node_init_body(x_ref, w_ref, b_ref, o_ref):
    acc = jnp.dot(x_ref[...], w_ref[...], preferred_element_type=jnp.float32)
    o_ref[...] = jax.nn.relu(acc + b_ref[...])


def _node_init(x, WnT, b_row):
    R = 2000
    return pl.pallas_call(
        _node_init_body,
        grid=(N // R,),
        in_specs=[
            pl.BlockSpec((R, D_IN), lambda i: (i, 0)),
            pl.BlockSpec((D_IN, H), lambda i: (0, 0)),
            pl.BlockSpec((1, H), lambda i: (0, 0)),
        ],
        out_specs=pl.BlockSpec((R, H), lambda i: (i, 0)),
        out_shape=jax.ShapeDtypeStruct((N, H), jnp.float32),
    )(x, WnT, b_row)


def _hid_body(w_ref, ea_ref, b_ref, o_ref):
    h = jnp.dot(w_ref[...], ea_ref[...], preferred_element_type=jnp.float32)
    o_ref[...] = jax.nn.relu(h + b_ref[...]).astype(jnp.bfloat16)


def _hid(W_e1, eaT, b_col):
    B = 1280
    return pl.pallas_call(
        _hid_body,
        grid=(E // B,),
        in_specs=[
            pl.BlockSpec((EH, D_EDGE), lambda i: (0, 0)),
            pl.BlockSpec((D_EDGE, B), lambda i: (0, i)),
            pl.BlockSpec((EH, 1), lambda i: (0, 0)),
        ],
        out_specs=pl.BlockSpec((EH, B), lambda i: (0, i)),
        out_shape=jax.ShapeDtypeStruct((EH, E), jnp.bfloat16),
    )(W_e1, eaT, b_col)


def _msg_body(w2_ref, b2t_ref, hidT_ref, xj_ref, o_ref):
    ewT = jnp.dot(w2_ref[...], hidT_ref[...], preferred_element_type=jnp.float32)
    ew3 = ewT.reshape(H, H, ewT.shape[-1])
    xjT = xj_ref[...].T
    msgT = jnp.sum(ew3 * xjT[:, None, :], axis=0)
    msgT = msgT + jnp.dot(b2t_ref[...], xjT, preferred_element_type=jnp.float32)
    o_ref[...] = msgT.T


def _msg(W_e2b, b2t, hidT, xj):
    B = 1280
    ne = xj.shape[0]
    return pl.pallas_call(
        _msg_body,
        grid=(ne // B,),
        in_specs=[
            pl.BlockSpec((H * H, EH), lambda i: (0, 0)),
            pl.BlockSpec((H, H), lambda i: (0, 0)),
            pl.BlockSpec((EH, B), lambda i: (0, i)),
            pl.BlockSpec((B, H), lambda i: (i, 0)),
        ],
        out_specs=pl.BlockSpec((B, H), lambda i: (i, 0)),
        out_shape=jax.ShapeDtypeStruct((ne, H), jnp.float32),
    )(W_e2b, b2t, hidT, xj)


def _msg_fused_body(w2_ref, b2t_ref, hidT_ref, gj_ref, xo_ref,
                    wr_ref, wir_ref, wiz_ref, win_ref, whr_ref, whz_ref,
                    whn_ref, bc_ref, bir_ref, biz_ref, bin_ref, bhr_ref,
                    bhz_ref, bhn_ref, omsg_ref, oxj_ref):
    xo = xo_ref[...]
    aggr = gj_ref[0] + gj_ref[1]
    conv = aggr + jnp.dot(xo, wr_ref[...], preferred_element_type=jnp.float32)
    m = jax.nn.relu(conv + bc_ref[...])
    i_r = jnp.dot(m, wir_ref[...], preferred_element_type=jnp.float32) + bir_ref[...]
    i_z = jnp.dot(m, wiz_ref[...], preferred_element_type=jnp.float32) + biz_ref[...]
    i_n = jnp.dot(m, win_ref[...], preferred_element_type=jnp.float32) + bin_ref[...]
    h_r = jnp.dot(xo, whr_ref[...], preferred_element_type=jnp.float32) + bhr_ref[...]
    h_z = jnp.dot(xo, whz_ref[...], preferred_element_type=jnp.float32) + bhz_ref[...]
    h_n = jnp.dot(xo, whn_ref[...], preferred_element_type=jnp.float32) + bhn_ref[...]
    r = jax.nn.sigmoid(i_r + h_r)
    z = jax.nn.sigmoid(i_z + h_z)
    ng = jnp.tanh(i_n + r * h_n)
    x = (1.0 - z) * ng + z * xo
    oxj_ref[...] = x
    ewT = jnp.dot(w2_ref[...], hidT_ref[...], preferred_element_type=jnp.float32)
    ew3 = ewT.reshape(H, H, ewT.shape[-1])
    xjT = x.T
    msgT = jnp.sum(ew3 * xjT[:, None, :], axis=0)
    msgT = msgT + jnp.dot(b2t_ref[...], xjT, preferred_element_type=jnp.float32)
    omsg_ref[...] = msgT.T


def _msg_fused(W_e2b, b2t, hidT, gj, xjold, mats, biases):
    B = 1280
    ne = xjold.shape[0]
    w_spec = pl.BlockSpec((H, H), lambda i: (0, 0))
    b_spec = pl.BlockSpec((1, H), lambda i: (0, 0))
    eb_spec = pl.BlockSpec((B, H), lambda i: (i, 0))
    return pl.pallas_call(
        _msg_fused_body,
        grid=(ne // B,),
        in_specs=[
            pl.BlockSpec((H * H, EH), lambda i: (0, 0)),
            pl.BlockSpec((H, H), lambda i: (0, 0)),
            pl.BlockSpec((EH, B), lambda i: (0, i)),
            pl.BlockSpec((NC, B, H), lambda i: (0, i, 0)),
            eb_spec,
        ] + [w_spec] * 7 + [b_spec] * 7,
        out_specs=[eb_spec, eb_spec],
        out_shape=[jax.ShapeDtypeStruct((ne, H), jnp.float32),
                   jax.ShapeDtypeStruct((ne, H), jnp.float32)],
    )(W_e2b, b2t, hidT, gj, xjold, *mats, *biases)


def _update_body(a2_ref, node_ref,
                 wr_ref, wir_ref, wiz_ref, win_ref, whr_ref, whz_ref, whn_ref,
                 bc_ref, bir_ref, biz_ref, bin_ref, bhr_ref, bhz_ref, bhn_ref,
                 o_ref):
    node = node_ref[...]
    aggr = a2_ref[0] + a2_ref[1]
    conv = aggr + jnp.dot(node, wr_ref[...], preferred_element_type=jnp.float32)
    m = jax.nn.relu(conv + bc_ref[...])
    i_r = jnp.dot(m, wir_ref[...], preferred_element_type=jnp.float32) + bir_ref[...]
    i_z = jnp.dot(m, wiz_ref[...], preferred_element_type=jnp.float32) + biz_ref[...]
    i_n = jnp.dot(m, win_ref[...], preferred_element_type=jnp.float32) + bin_ref[...]
    h_r = jnp.dot(node, whr_ref[...], preferred_element_type=jnp.float32) + bhr_ref[...]
    h_z = jnp.dot(node, whz_ref[...], preferred_element_type=jnp.float32) + bhz_ref[...]
    h_n = jnp.dot(node, whn_ref[...], preferred_element_type=jnp.float32) + bhn_ref[...]
    r = jax.nn.sigmoid(i_r + h_r)
    z = jax.nn.sigmoid(i_z + h_z)
    ng = jnp.tanh(i_n + r * h_n)
    o_ref[...] = (1.0 - z) * ng + z * node


def _update(a2, node, mats, biases):
    R = 2000
    w_spec = pl.BlockSpec((H, H), lambda i: (0, 0))
    b_spec = pl.BlockSpec((1, H), lambda i: (0, 0))
    return pl.pallas_call(
        _update_body,
        grid=(N // R,),
        in_specs=[
            pl.BlockSpec((NC, R, H), lambda i: (0, i, 0)),
            pl.BlockSpec((R, H), lambda i: (i, 0)),
        ] + [w_spec] * 7 + [b_spec] * 7,
        out_specs=pl.BlockSpec((R, H), lambda i: (i, 0)),
        out_shape=jax.ShapeDtypeStruct((N, H), jnp.float32),
    )(a2, node, *mats, *biases)


_sc_cache = {}


def _sc_kernels(nsup):
    if nsup in _sc_cache:
        return _sc_cache[nsup]

    assert nsup % NW == 0, "edges must split evenly across the 32 subcores"
    ne = nsup * SUP
    supw = nsup // NW
    mesh = plsc.VectorSubcoreMesh(core_axis_name="c", subcore_axis_name="s",
                                  num_cores=NC, num_subcores=NS)
    cp = pltpu.CompilerParams(use_tc_tiling_on_sc=False)

    @functools.partial(
        pl.kernel,
        out_type=jax.ShapeDtypeStruct((ne, H), jnp.float32),
        mesh=mesh,
        compiler_params=cp,
        scratch_types=[
            pltpu.VMEM((2, NSTR, IB), jnp.int32),
            pltpu.VMEM((2, SUP, H), jnp.float32),
            pltpu.SemaphoreType.DMA,
            pltpu.SemaphoreType.DMA,
            pltpu.SemaphoreType.DMA,
        ],
    )
    def gather_k(node_hbm, src2_hbm, out_hbm, idx_v, rows_v, gsem, w0, w1):
        wid = lax.axis_index("s") * NC + lax.axis_index("c")
        wsems = (w0, w1)
        wb = [None, None]
        for i in range(supw):
            b = i % 2
            s = wid + i * NW
            if wb[b] is not None:
                wb[b].wait()
                wb[b] = None
            pltpu.sync_copy(src2_hbm.at[pl.ds(s * NSTR, NSTR)], idx_v.at[b])
            cps = [pltpu.async_copy(node_hbm.at[idx_v.at[b].at[j]],
                                    rows_v.at[b].at[pl.ds(j * IB, IB)], gsem)
                   for j in range(NSTR)]
            for cp_ in cps:
                cp_.wait()
            wb[b] = pltpu.async_copy(rows_v.at[b],
                                     out_hbm.at[pl.ds(s * SUP, SUP)], wsems[b])
        for cp_ in wb:
            if cp_ is not None:
                cp_.wait()

    @functools.partial(
        pl.kernel,
        out_type=jax.ShapeDtypeStruct((NC, N, H), jnp.float32),
        mesh=mesh,
        compiler_params=cp,
        scratch_types=[
            pltpu.VMEM((2, NSTR, IB), jnp.int32),
            pltpu.VMEM((2, SUP, H), jnp.float32),
            pltpu.VMEM_SHARED((N, H), jnp.float32),
            pltpu.SemaphoreType.DMA,
            pltpu.SemaphoreType.DMA,
            pltpu.SemaphoreType.DMA,
        ],
    )
    def scatter_k(msg_hbm, dst2_hbm, zero_hbm, out_hbm, idx_v, row_v, acc_sh,
                  ssem, l0, l1):
        cid = lax.axis_index("c")
        sid = lax.axis_index("s")
        wid = sid * NC + cid
        r0 = sid * ROWS_PER_SUB
        pltpu.sync_copy(zero_hbm.at[pl.ds(r0, ROWS_PER_SUB)],
                        acc_sh.at[pl.ds(r0, ROWS_PER_SUB)])
        plsc.subcore_barrier()

        lsems = (l0, l1)

        def _start_loads(i):
            b = i % 2
            s = wid + i * NW
            return (pltpu.async_copy(dst2_hbm.at[pl.ds(s * NSTR, NSTR)],
                                     idx_v.at[b], lsems[b]),
                    pltpu.async_copy(msg_hbm.at[pl.ds(s * SUP, SUP)],
                                     row_v.at[b], lsems[b]))

        pending = _start_loads(0)
        for i in range(supw):
            b = i % 2
            for cp_ in pending:
                cp_.wait()
            if i + 1 < supw:
                pending = _start_loads(i + 1)
            cps = [pltpu.async_copy(row_v.at[b].at[pl.ds(j * IB, IB)],
                                    acc_sh.at[idx_v.at[b].at[j]], ssem, add=True)
                   for j in range(NSTR)]
            for cp_ in cps:
                cp_.wait()

        plsc.subcore_barrier()
        pltpu.sync_copy(acc_sh.at[pl.ds(r0, ROWS_PER_SUB)],
                        out_hbm.at[cid].at[pl.ds(r0, ROWS_PER_SUB)])

    @functools.partial(
        pl.kernel,
        out_type=[jax.ShapeDtypeStruct((NC, N, H), jnp.float32),
                  jax.ShapeDtypeStruct((NC, ne, H), jnp.float32)],
        mesh=mesh,
        compiler_params=cp,
        scratch_types=[
            pltpu.VMEM((2, NSTR, IB), jnp.int32),
            pltpu.VMEM((2, SUP, H), jnp.float32),
            pltpu.VMEM_SHARED((N, H), jnp.float32),
            pltpu.SemaphoreType.DMA,
            pltpu.SemaphoreType.DMA,
            pltpu.SemaphoreType.DMA,
        ],
    )
    def scatter_gj_k(msg_hbm, dst2_hbm, src2_hbm, zero_hbm, out_hbm, gj_hbm,
                     idx_v, row_v, acc_sh, ssem, l0, l1):
        cid = lax.axis_index("c")
        sid = lax.axis_index("s")
        wid = sid * NC + cid
        r0 = sid * ROWS_PER_SUB
        pltpu.sync_copy(zero_hbm.at[pl.ds(r0, ROWS_PER_SUB)],
                        acc_sh.at[pl.ds(r0, ROWS_PER_SUB)])
        plsc.subcore_barrier()

        lsems = (l0, l1)

        def _start_loads(i):
            b = i % 2
            s = wid + i * NW
            return (pltpu.async_copy(dst2_hbm.at[pl.ds(s * NSTR, NSTR)],
                                     idx_v.at[b], lsems[b]),
                    pltpu.async_copy(msg_hbm.at[pl.ds(s * SUP, SUP)],
                                     row_v.at[b], lsems[b]))

        pending = _start_loads(0)
        for i in range(supw):
            b = i % 2
            for cp_ in pending:
                cp_.wait()
            if i + 1 < supw:
                pending = _start_loads(i + 1)
            cps = [pltpu.async_copy(row_v.at[b].at[pl.ds(j * IB, IB)],
                                    acc_sh.at[idx_v.at[b].at[j]], ssem, add=True)
                   for j in range(NSTR)]
            for cp_ in cps:
                cp_.wait()

        plsc.subcore_barrier()
        pltpu.sync_copy(acc_sh.at[pl.ds(r0, ROWS_PER_SUB)],
                        out_hbm.at[cid].at[pl.ds(r0, ROWS_PER_SUB)])
        plsc.subcore_barrier()
        table = out_hbm.at[cid]
        gj_c = gj_hbm.at[cid]
        for i in range(nsup // NS):
            b = i % 2
            s = sid + i * NS
            pltpu.sync_copy(src2_hbm.at[pl.ds(s * NSTR, NSTR)], idx_v.at[b])
            cps = [pltpu.async_copy(table.at[idx_v.at[b].at[j]],
                                    row_v.at[b].at[pl.ds(j * IB, IB)], ssem)
                   for j in range(NSTR)]
            for cp_ in cps:
                cp_.wait()
            pltpu.sync_copy(row_v.at[b], gj_c.at[pl.ds(s * SUP, SUP)])

    _sc_cache[nsup] = (gather_k, scatter_k, scatter_gj_k)
    return _sc_cache[nsup]


def _sc_gather(node, src2):
    gather_k, _, _ = _sc_kernels(src2.shape[0] // NSTR)
    return gather_k(node, src2)


def _sc_scatter_add(msg, dst2, zeros_nh):
    _, scatter_k, _ = _sc_kernels(dst2.shape[0] // NSTR)
    return scatter_k(msg, dst2, zeros_nh)


def _sc_scatter_gj(msg, dst2, src2, zeros_nh):
    _, _, scatter_gj_k = _sc_kernels(dst2.shape[0] // NSTR)
    return scatter_gj_k(msg, dst2, src2, zeros_nh)


def kernel(x, edge_index, edge_attr, W_node, b_node, W_e1, b_e1, W_e2, b_e2,
           W_root, b_conv, W_ih, W_hh, b_ih, b_hh):
    src2 = edge_index[0].reshape(E // IB, IB)
    dst2 = edge_index[1].reshape(E // IB, IB)

    WnT = W_node.T
    eaT = edge_attr.T
    b_e1c = b_e1.reshape(EH, 1)
    W_e2b = W_e2.astype(jnp.bfloat16)
    b2t = b_e2.reshape(H, H).T
    zeros_nh = jnp.zeros((N, H), jnp.float32)

    mats = (
        W_root.T,
        W_ih[0:H].T, W_ih[H:2 * H].T, W_ih[2 * H:3 * H].T,
        W_hh[0:H].T, W_hh[H:2 * H].T, W_hh[2 * H:3 * H].T,
    )
    biases = (
        b_conv.reshape(1, H),
        b_ih[0:H].reshape(1, H), b_ih[H:2 * H].reshape(1, H),
        b_ih[2 * H:3 * H].reshape(1, H),
        b_hh[0:H].reshape(1, H), b_hh[H:2 * H].reshape(1, H),
        b_hh[2 * H:3 * H].reshape(1, H),
    )

    node = _node_init(x, WnT, b_node.reshape(1, H))
    hidT = _hid(W_e1, eaT, b_e1c)

    xj = _sc_gather(node, src2)
    msg = _msg(W_e2b, b2t, hidT, xj)
    for step in range(STEPS):
        if step == STEPS - 1:
            a2 = _sc_scatter_add(msg, dst2, zeros_nh)
        else:
            a2, gj = _sc_scatter_gj(msg, dst2, src2, zeros_nh)
        node = _update(a2, node, mats, biases)
        if step < STEPS - 1:
            msg, xj = _msg_fused(W_e2b, b2t, hidT, gj, xj, mats, biases)

    return node


# speedup vs baseline: 1.2500x; 1.2500x over previous
import functools

import jax
import jax.numpy as jnp
from jax import lax
from jax.experimental import pallas as pl
from jax.experimental.pallas import tpu as pltpu
from jax.experimental.pallas import tpu_sc as plsc

N = 10000
E = 160000
D_IN = 128
D_EDGE = 16
H = 32
EH = 128
STEPS = 3

NC = 2
NS = 16
NW = NC * NS
IB = 125
NSTR = 8
SUP = IB * NSTR
NSUP = E // SUP
SUPW = NSUP // NW
ROWS_PER_SUB = N // NS


def _node_init_body(x_ref, w_ref, b_ref, o_ref):
    acc = jnp.dot(x_ref[...], w_ref[...], preferred_element_type=jnp.float32)
    o_ref[...] = jax.nn.relu(acc + b_ref[...])


def _node_init(x, WnT, b_row):
    R = 2000
    return pl.pallas_call(
        _node_init_body,
        grid=(N // R,),
        in_specs=[
            pl.BlockSpec((R, D_IN), lambda i: (i, 0)),
            pl.BlockSpec((D_IN, H), lambda i: (0, 0)),
            pl.BlockSpec((1, H), lambda i: (0, 0)),
        ],
        out_specs=pl.BlockSpec((R, H), lambda i: (i, 0)),
        out_shape=jax.ShapeDtypeStruct((N, H), jnp.float32),
    )(x, WnT, b_row)


def _hid_body(w_ref, ea_ref, b_ref, o_ref):
    h = jnp.dot(w_ref[...], ea_ref[...], preferred_element_type=jnp.float32)
    o_ref[...] = jax.nn.relu(h + b_ref[...]).astype(jnp.bfloat16)


def _hid(W_e1, eaT, b_col):
    B = 1280
    return pl.pallas_call(
        _hid_body,
        grid=(E // B,),
        in_specs=[
            pl.BlockSpec((EH, D_EDGE), lambda i: (0, 0)),
            pl.BlockSpec((D_EDGE, B), lambda i: (0, i)),
            pl.BlockSpec((EH, 1), lambda i: (0, 0)),
        ],
        out_specs=pl.BlockSpec((EH, B), lambda i: (0, i)),
        out_shape=jax.ShapeDtypeStruct((EH, E), jnp.bfloat16),
    )(W_e1, eaT, b_col)


def _msg_body(w2_ref, b2t_ref, hidT_ref, xj_ref, o_ref):
    ewT = jnp.dot(w2_ref[...], hidT_ref[...], preferred_element_type=jnp.float32)
    ew3 = ewT.reshape(H, H, ewT.shape[-1])
    xjT = xj_ref[...].T
    msgT = jnp.sum(ew3 * xjT[:, None, :], axis=0)
    msgT = msgT + jnp.dot(b2t_ref[...], xjT, preferred_element_type=jnp.float32)
    o_ref[...] = msgT.T


def _msg(W_e2b, b2t, hidT, xj):
    B = 1280
    ne = xj.shape[0]
    return pl.pallas_call(
        _msg_body,
        grid=(ne // B,),
        in_specs=[
            pl.BlockSpec((H * H, EH), lambda i: (0, 0)),
            pl.BlockSpec((H, H), lambda i: (0, 0)),
            pl.BlockSpec((EH, B), lambda i: (0, i)),
            pl.BlockSpec((B, H), lambda i: (i, 0)),
        ],
        out_specs=pl.BlockSpec((B, H), lambda i: (i, 0)),
        out_shape=jax.ShapeDtypeStruct((ne, H), jnp.float32),
    )(W_e2b, b2t, hidT, xj)


def _update_body(a2_ref, node_ref,
                 wr_ref, wir_ref, wiz_ref, win_ref, whr_ref, whz_ref, whn_ref,
                 bc_ref, bir_ref, biz_ref, bin_ref, bhr_ref, bhz_ref, bhn_ref,
                 o_ref):
    node = node_ref[...]
    aggr = a2_ref[0] + a2_ref[1]
    conv = aggr + jnp.dot(node, wr_ref[...], preferred_element_type=jnp.float32)
    m = jax.nn.relu(conv + bc_ref[...])
    i_r = jnp.dot(m, wir_ref[...], preferred_element_type=jnp.float32) + bir_ref[...]
    i_z = jnp.dot(m, wiz_ref[...], preferred_element_type=jnp.float32) + biz_ref[...]
    i_n = jnp.dot(m, win_ref[...], preferred_element_type=jnp.float32) + bin_ref[...]
    h_r = jnp.dot(node, whr_ref[...], preferred_element_type=jnp.float32) + bhr_ref[...]
    h_z = jnp.dot(node, whz_ref[...], preferred_element_type=jnp.float32) + bhz_ref[...]
    h_n = jnp.dot(node, whn_ref[...], preferred_element_type=jnp.float32) + bhn_ref[...]
    r = jax.nn.sigmoid(i_r + h_r)
    z = jax.nn.sigmoid(i_z + h_z)
    ng = jnp.tanh(i_n + r * h_n)
    o_ref[...] = (1.0 - z) * ng + z * node


def _update(a2, node, mats, biases):
    R = 2000
    w_spec = pl.BlockSpec((H, H), lambda i: (0, 0))
    b_spec = pl.BlockSpec((1, H), lambda i: (0, 0))
    return pl.pallas_call(
        _update_body,
        grid=(N // R,),
        in_specs=[
            pl.BlockSpec((NC, R, H), lambda i: (0, i, 0)),
            pl.BlockSpec((R, H), lambda i: (i, 0)),
        ] + [w_spec] * 7 + [b_spec] * 7,
        out_specs=pl.BlockSpec((R, H), lambda i: (i, 0)),
        out_shape=jax.ShapeDtypeStruct((N, H), jnp.float32),
    )(a2, node, *mats, *biases)


_sc_cache = {}


def _sc_kernels(nsup):
    if nsup in _sc_cache:
        return _sc_cache[nsup]

    assert nsup % NW == 0, "edges must split evenly across the 32 subcores"
    ne = nsup * SUP
    supw = nsup // NW
    mesh = plsc.VectorSubcoreMesh(core_axis_name="c", subcore_axis_name="s",
                                  num_cores=NC, num_subcores=NS)
    cp = pltpu.CompilerParams(use_tc_tiling_on_sc=False)

    @functools.partial(
        pl.kernel,
        out_type=jax.ShapeDtypeStruct((ne, H), jnp.float32),
        mesh=mesh,
        compiler_params=cp,
        scratch_types=[
            pltpu.VMEM((2, NSTR, IB), jnp.int32),
            pltpu.VMEM((2, SUP, H), jnp.float32),
            pltpu.SemaphoreType.DMA,
            pltpu.SemaphoreType.DMA,
            pltpu.SemaphoreType.DMA,
        ],
    )
    def gather_k(node_hbm, src2_hbm, out_hbm, idx_v, rows_v, gsem, w0, w1):
        wid = lax.axis_index("s") * NC + lax.axis_index("c")
        wsems = (w0, w1)
        wb = [None, None]
        for i in range(supw):
            b = i % 2
            s = wid + i * NW
            if wb[b] is not None:
                wb[b].wait()
                wb[b] = None
            pltpu.sync_copy(src2_hbm.at[pl.ds(s * NSTR, NSTR)], idx_v.at[b])
            cps = [pltpu.async_copy(node_hbm.at[idx_v.at[b].at[j]],
                                    rows_v.at[b].at[pl.ds(j * IB, IB)], gsem)
                   for j in range(NSTR)]
            for cp_ in cps:
                cp_.wait()
            wb[b] = pltpu.async_copy(rows_v.at[b],
                                     out_hbm.at[pl.ds(s * SUP, SUP)], wsems[b])
        for cp_ in wb:
            if cp_ is not None:
                cp_.wait()

    @functools.partial(
        pl.kernel,
        out_type=jax.ShapeDtypeStruct((NC, N, H), jnp.float32),
        mesh=mesh,
        compiler_params=cp,
        scratch_types=[
            pltpu.VMEM((2, NSTR, IB), jnp.int32),
            pltpu.VMEM((2, SUP, H), jnp.float32),
            pltpu.VMEM_SHARED((N, H), jnp.float32),
            pltpu.SemaphoreType.DMA,
            pltpu.SemaphoreType.DMA,
            pltpu.SemaphoreType.DMA,
        ],
    )
    def scatter_k(msg_hbm, dst2_hbm, zero_hbm, out_hbm, idx_v, row_v, acc_sh,
                  ssem, l0, l1):
        cid = lax.axis_index("c")
        sid = lax.axis_index("s")
        wid = sid * NC + cid
        r0 = sid * ROWS_PER_SUB
        pltpu.sync_copy(zero_hbm.at[pl.ds(r0, ROWS_PER_SUB)],
                        acc_sh.at[pl.ds(r0, ROWS_PER_SUB)])
        plsc.subcore_barrier()

        lsems = (l0, l1)

        def _start_loads(i):
            b = i % 2
            s = wid + i * NW
            return (pltpu.async_copy(dst2_hbm.at[pl.ds(s * NSTR, NSTR)],
                                     idx_v.at[b], lsems[b]),
                    pltpu.async_copy(msg_hbm.at[pl.ds(s * SUP, SUP)],
                                     row_v.at[b], lsems[b]))

        pending = _start_loads(0)
        for i in range(supw):
            b = i % 2
            for cp_ in pending:
                cp_.wait()
            if i + 1 < supw:
                pending = _start_loads(i + 1)
            cps = [pltpu.async_copy(row_v.at[b].at[pl.ds(j * IB, IB)],
                                    acc_sh.at[idx_v.at[b].at[j]], ssem, add=True)
                   for j in range(NSTR)]
            for cp_ in cps:
                cp_.wait()

        plsc.subcore_barrier()
        pltpu.sync_copy(acc_sh.at[pl.ds(r0, ROWS_PER_SUB)],
                        out_hbm.at[cid].at[pl.ds(r0, ROWS_PER_SUB)])

    _sc_cache[nsup] = (gather_k, scatter_k)
    return gather_k, scatter_k


def _sc_gather(node, src2):
    gather_k, _ = _sc_kernels(src2.shape[0] // NSTR)
    return gather_k(node, src2)


def _sc_scatter_add(msg, dst2, zeros_nh):
    _, scatter_k = _sc_kernels(dst2.shape[0] // NSTR)
    return scatter_k(msg, dst2, zeros_nh)


def kernel(x, edge_index, edge_attr, W_node, b_node, W_e1, b_e1, W_e2, b_e2,
           W_root, b_conv, W_ih, W_hh, b_ih, b_hh):
    src2 = edge_index[0].reshape(E // IB, IB)
    dst2 = edge_index[1].reshape(E // IB, IB)

    WnT = W_node.T
    eaT = edge_attr.T
    b_e1c = b_e1.reshape(EH, 1)
    W_e2b = W_e2.astype(jnp.bfloat16)
    b2t = b_e2.reshape(H, H).T
    zeros_nh = jnp.zeros((N, H), jnp.float32)

    mats = (
        W_root.T,
        W_ih[0:H].T, W_ih[H:2 * H].T, W_ih[2 * H:3 * H].T,
        W_hh[0:H].T, W_hh[H:2 * H].T, W_hh[2 * H:3 * H].T,
    )
    biases = (
        b_conv.reshape(1, H),
        b_ih[0:H].reshape(1, H), b_ih[H:2 * H].reshape(1, H),
        b_ih[2 * H:3 * H].reshape(1, H),
        b_hh[0:H].reshape(1, H), b_hh[H:2 * H].reshape(1, H),
        b_hh[2 * H:3 * H].reshape(1, H),
    )

    node = _node_init(x, WnT, b_node.reshape(1, H))
    hidT = _hid(W_e1, eaT, b_e1c)

    for _ in range(STEPS):
        xj = _sc_gather(node, src2)
        msg = _msg(W_e2b, b2t, hidT, xj)
        a2 = _sc_scatter_add(msg, dst2, zeros_nh)
        node = _update(a2, node, mats, biases)

    return node
